# Initial kernel scaffold; baseline (speedup 1.0000x reference)
import jax
import jax.numpy as jnp
from jax.experimental import pallas as pl


def kernel(x, table):
    raise NotImplementedError("write your pallas kernel here")


# speedup vs baseline: 4.2287x; 4.2287x over previous
import functools

import jax
import jax.numpy as jnp
from jax import lax
from jax.experimental import pallas as pl
from jax.experimental.pallas import tpu as pltpu
from jax.experimental.pallas import tpu_sc as plsc

BATCH = 16384
EMBED = 128
FIELD0 = 100000
NC = 2
NS = 16
NW = NC * NS
BW = BATCH // NW
CH = 128
NCHUNK = BW // CH
LANES = 16

_mesh = plsc.VectorSubcoreMesh(core_axis_name="c", subcore_axis_name="s")


@functools.partial(
    pl.kernel,
    mesh=_mesh,
    out_type=jax.ShapeDtypeStruct((BATCH,), jnp.float32),
    scratch_types=[
        pltpu.VMEM((BW,), jnp.int32),
        pltpu.VMEM((BW,), jnp.int32),
        pltpu.VMEM((CH, EMBED), jnp.float32),
        pltpu.VMEM((CH, EMBED), jnp.float32),
        pltpu.VMEM((BW,), jnp.float32),
        pltpu.VMEM((96,), jnp.float32),
        pltpu.SemaphoreType.DMA,
    ],
)
def _cmf_fwd(iu_hbm, ii_hbm, table_hbm, out_hbm, iu_v, ii_v, uv, vv, ov, shf, sem):
    wid = lax.axis_index("s") * NC + lax.axis_index("c")
    base = wid * BW
    pltpu.sync_copy(iu_hbm.at[pl.ds(base, BW)], iu_v)
    pltpu.sync_copy(ii_hbm.at[pl.ds(base, BW)], ii_v)

    lanes_iota = lax.iota(jnp.int32, LANES)
    bit_masks = [((lanes_iota >> k) & 1) == 1 for k in range(4)]

    def hshift(x, s, center):
        shf[pl.ds(center, LANES)] = x
        return shf[pl.ds(center - s, LANES)]

    def merge(lo, hi, k):
        hi2 = hi + hshift(hi, 1 << k, 16)
        lo2 = lo + hshift(lo, -(1 << k), 56)
        return jnp.where(bit_masks[k], hi2, lo2)

    for c in range(NCHUNK):
        pltpu.async_copy(table_hbm.at[iu_v.at[pl.ds(c * CH, CH)]], uv, sem).wait()
        pltpu.async_copy(table_hbm.at[ii_v.at[pl.ds(c * CH, CH)]], vv, sem).wait()

        def group(g, _, c=c):
            vecs = []
            for e in range(LANES):
                b = g * LANES + e
                acc = uv[b, pl.ds(0, LANES)] * vv[b, pl.ds(0, LANES)]
                for dj in range(1, EMBED // LANES):
                    acc = acc + uv[b, pl.ds(dj * LANES, LANES)] * vv[b, pl.ds(dj * LANES, LANES)]
                vecs.append(acc)
            for k in range(4):
                vecs = [merge(vecs[2 * i], vecs[2 * i + 1], k)
                        for i in range(len(vecs) // 2)]
            ov[pl.ds(c * CH + g * LANES, LANES)] = vecs[0]
            return _

        lax.fori_loop(0, CH // LANES, group, 0)

    def sig(j, _):
        z = ov[pl.ds(j * LANES, LANES)]
        ov[pl.ds(j * LANES, LANES)] = 1.0 / (1.0 + jnp.exp(-z))
        return _

    lax.fori_loop(0, BW // LANES, sig, 0)
    pltpu.sync_copy(ov, out_hbm.at[pl.ds(base, BW)])


def kernel(x, table):
    x = x.astype(jnp.int32)
    iu = x[:, 0]
    ii = x[:, 1] + jnp.int32(FIELD0)
    return _cmf_fwd(iu, ii, table)


# speedup vs baseline: 5.2416x; 1.2395x over previous
import functools

import jax
import jax.numpy as jnp
from jax import lax
from jax.experimental import pallas as pl
from jax.experimental.pallas import tpu as pltpu
from jax.experimental.pallas import tpu_sc as plsc

BATCH = 16384
EMBED = 128
FIELD0 = 100000
NC = 2
NS = 16
NW = NC * NS
BW = BATCH // NW
CH = 128
NCHUNK = BW // CH
LANES = 16

_mesh = plsc.VectorSubcoreMesh(core_axis_name="c", subcore_axis_name="s")


@functools.partial(
    pl.kernel,
    mesh=_mesh,
    out_type=jax.ShapeDtypeStruct((BATCH,), jnp.float32),
    scratch_types=[
        pltpu.VMEM((BW,), jnp.int32),
        pltpu.VMEM((BW,), jnp.int32),
        pltpu.VMEM((CH, EMBED), jnp.float32),
        pltpu.VMEM((CH, EMBED), jnp.float32),
        pltpu.VMEM((CH, EMBED), jnp.float32),
        pltpu.VMEM((CH, EMBED), jnp.float32),
        pltpu.VMEM((BW,), jnp.float32),
        pltpu.VMEM((96,), jnp.float32),
        pltpu.SemaphoreType.DMA,
    ],
)
def _cmf_fwd(iu_hbm, ii_hbm, table_hbm, out_hbm,
             iu_v, ii_v, uv0, uv1, vv0, vv1, ov, shf, sem):
    ubufs = (uv0, uv1)
    vbufs = (vv0, vv1)
    wid = lax.axis_index("s") * NC + lax.axis_index("c")
    base = wid * BW
    pltpu.sync_copy(iu_hbm.at[pl.ds(base, BW)], iu_v)
    pltpu.sync_copy(ii_hbm.at[pl.ds(base, BW)], ii_v)

    lanes_iota = lax.iota(jnp.int32, LANES)
    bit_masks = [((lanes_iota >> k) & 1) == 1 for k in range(4)]

    def hshift(x, s, center):
        shf[pl.ds(center, LANES)] = x
        return shf[pl.ds(center - s, LANES)]

    def merge(lo, hi, k):
        hi2 = hi + hshift(hi, 1 << k, 16)
        lo2 = lo + hshift(lo, -(1 << k), 56)
        return jnp.where(bit_masks[k], hi2, lo2)

    def gather(c):
        slot = c % 2
        cu = pltpu.async_copy(
            table_hbm.at[iu_v.at[pl.ds(c * CH, CH)]], ubufs[slot], sem)
        cv = pltpu.async_copy(
            table_hbm.at[ii_v.at[pl.ds(c * CH, CH)]], vbufs[slot], sem)
        return cu, cv

    pending = gather(0)
    for c in range(NCHUNK):
        cu, cv = pending
        cu.wait()
        cv.wait()
        if c + 1 < NCHUNK:
            pending = gather(c + 1)
        uv = ubufs[c % 2]
        vv = vbufs[c % 2]

        def group(g, _, c=c, uv=uv, vv=vv):
            vecs = []
            for e in range(LANES):
                b = g * LANES + e
                acc = uv[b, pl.ds(0, LANES)] * vv[b, pl.ds(0, LANES)]
                for dj in range(1, EMBED // LANES):
                    acc = acc + uv[b, pl.ds(dj * LANES, LANES)] * vv[b, pl.ds(dj * LANES, LANES)]
                vecs.append(acc)
            for k in range(4):
                vecs = [merge(vecs[2 * i], vecs[2 * i + 1], k)
                        for i in range(len(vecs) // 2)]
            ov[pl.ds(c * CH + g * LANES, LANES)] = vecs[0]
            return _

        lax.fori_loop(0, CH // LANES, group, 0)

    def sig(j, _):
        z = ov[pl.ds(j * LANES, LANES)]
        ov[pl.ds(j * LANES, LANES)] = 1.0 / (1.0 + jnp.exp(-z))
        return _

    lax.fori_loop(0, BW // LANES, sig, 0)
    pltpu.sync_copy(ov, out_hbm.at[pl.ds(base, BW)])


def kernel(x, table):
    x = x.astype(jnp.int32)
    iu = x[:, 0]
    ii = x[:, 1] + jnp.int32(FIELD0)
    return _cmf_fwd(iu, ii, table)
